# Initial kernel scaffold; baseline (speedup 1.0000x reference)
import jax
import jax.numpy as jnp
from jax.experimental import pallas as pl


def kernel(x, edge_index, P_vec, W1, b1, W2, b2):
    raise NotImplementedError("write your pallas kernel here")


# speedup vs baseline: 24.4972x; 24.4972x over previous
import functools

import jax
import jax.numpy as jnp
from jax import lax
from jax.experimental import pallas as pl
from jax.experimental.pallas import tpu as pltpu
from jax.experimental.pallas import tpu_sc as plsc

N = 10000
E = 320000
F_IN = 128
H = 32
O = 8
O_PAD = 16

NC = 2
NS = 16
NW = NC * NS
CHUNK = 128
CPT = 80
EPT = CPT * CHUNK
E_PAD = NW * EPT
REAL_CHUNKS = E // CHUNK
N_PAD = 10240
NPT = N_PAD // NS
DPAD = N_PAD
DPT = DPAD // NS

_mesh = plsc.VectorSubcoreMesh(
    core_axis_name="c", subcore_axis_name="s", num_cores=NC, num_subcores=NS)


def _wid():
    return lax.axis_index("c") * NS + lax.axis_index("s")


@functools.partial(
    pl.kernel,
    out_type=(
        jax.ShapeDtypeStruct((E_PAD // CHUNK, CHUNK), jnp.float32),
        jax.ShapeDtypeStruct((NC, DPAD), jnp.float32),
    ),
    mesh=_mesh,
    compiler_params=pltpu.CompilerParams(use_tc_tiling_on_sc=False),
    scratch_types=[
        pltpu.VMEM((CPT, CHUNK), jnp.float32),
        pltpu.VMEM((CPT, CHUNK), jnp.int32),
        pltpu.VMEM((CPT, CHUNK), jnp.float32),
        pltpu.VMEM((DPT,), jnp.float32),
        pltpu.VMEM_SHARED((DPAD,), jnp.float32),
    ],
)
def _deg_kernel(p2d, col2d, w2d, degp, p_v, col_v, w_v, zbuf, deg_sp):
    cid = lax.axis_index("c")
    sid = lax.axis_index("s")
    wid = cid * NS + sid
    pltpu.sync_copy(p2d.at[pl.ds(wid * CPT, CPT)], p_v)
    pltpu.sync_copy(col2d.at[pl.ds(wid * CPT, CPT)], col_v)

    def compute(i, _):
        scale = jnp.where(wid * CPT + i < REAL_CHUNKS, 1.0, 0.0)
        for c in range(CHUNK // 16):
            sl = pl.ds(c * 16, 16)
            p = p_v[i, sl]
            w_v[i, sl] = scale / (1.0 + jnp.exp(-p))
        return _
    lax.fori_loop(0, CPT, compute, None)
    pltpu.sync_copy(w_v, w2d.at[pl.ds(wid * CPT, CPT)])

    def zero(j, _):
        zbuf[pl.ds(j * 16, 16)] = jnp.zeros((16,), jnp.float32)
        return _
    lax.fori_loop(0, DPT // 16, zero, None)
    pltpu.sync_copy(zbuf, deg_sp.at[pl.ds(sid * DPT, DPT)])
    plsc.subcore_barrier()

    def scatter(i, _):
        pltpu.sync_copy(w_v.at[i], deg_sp.at[col_v.at[i]], add=True)
        return _
    lax.fori_loop(0, CPT, scatter, None)
    plsc.subcore_barrier()
    pltpu.sync_copy(deg_sp.at[pl.ds(sid * DPT, DPT)],
                    degp.at[cid, pl.ds(sid * DPT, DPT)])


def _make_mp_kernel(F):
    nz = 5

    @functools.partial(
        pl.kernel,
        out_type=jax.ShapeDtypeStruct((NC, N_PAD, F), jnp.float32),
        mesh=_mesh,
        compiler_params=pltpu.CompilerParams(use_tc_tiling_on_sc=False),
        scratch_types=[
            pltpu.VMEM((CPT, CHUNK), jnp.int32),
            pltpu.VMEM((CPT, CHUNK), jnp.int32),
            pltpu.VMEM((CPT, CHUNK), jnp.float32),
            pltpu.VMEM((CHUNK, F), jnp.float32),
            pltpu.VMEM((NPT // nz, F), jnp.float32),
            pltpu.VMEM_SHARED((N_PAD, F), jnp.float32),
        ],
    )
    def _mp(t_hbm, row2d, col2d, w2d, agg_out, row_v, col_v, w_v, gbuf,
            zbuf, agg_sp):
        cid = lax.axis_index("c")
        sid = lax.axis_index("s")
        wid = cid * NS + sid
        pltpu.sync_copy(row2d.at[pl.ds(wid * CPT, CPT)], row_v)
        pltpu.sync_copy(col2d.at[pl.ds(wid * CPT, CPT)], col_v)
        pltpu.sync_copy(w2d.at[pl.ds(wid * CPT, CPT)], w_v)

        @pl.when(cid == 0)
        def _():
            pltpu.sync_copy(t_hbm.at[pl.ds(sid * NPT, NPT)],
                            agg_sp.at[pl.ds(sid * NPT, NPT)])

        @pl.when(cid != 0)
        def _():
            def zero(j, _):
                for c in range(F // 16):
                    zbuf[j, pl.ds(c * 16, 16)] = jnp.zeros((16,), jnp.float32)
                return _
            lax.fori_loop(0, NPT // nz, zero, None)
            for k in range(nz):
                pltpu.sync_copy(
                    zbuf, agg_sp.at[pl.ds(sid * NPT + k * (NPT // nz),
                                          NPT // nz)])
        plsc.subcore_barrier()

        def chunk(i, _):
            pltpu.sync_copy(t_hbm.at[row_v.at[i]], gbuf)
            for g in range(CHUNK // 16):
                wv = w_v[i, pl.ds(g * 16, 16)]
                for ee in range(16):
                    e = g * 16 + ee
                    bc = jnp.full((16,), wv[ee], jnp.float32)
                    for c in range(F // 16):
                        sl = pl.ds(c * 16, 16)
                        gbuf[e, sl] = gbuf[e, sl] * bc
            pltpu.sync_copy(gbuf, agg_sp.at[col_v.at[i]], add=True)
            return _
        lax.fori_loop(0, CPT, chunk, None)
        plsc.subcore_barrier()
        pltpu.sync_copy(agg_sp.at[pl.ds(sid * NPT, NPT)],
                        agg_out.at[cid, pl.ds(sid * NPT, NPT)])

    return _mp


_mp1 = _make_mp_kernel(H)
_mp2 = _make_mp_kernel(O_PAD)


_BN = 2048


def _tc1_body(x_ref, w1_ref, deg_ref, t1_ref, s_ref):
    d = deg_ref[:, 0] + deg_ref[:, 1] + 1.0
    s = lax.rsqrt(d)
    sup = jnp.dot(x_ref[...], w1_ref[...], preferred_element_type=jnp.float32)
    t1_ref[...] = s[:, None] * sup
    s_ref[...] = s[:, None]


def _tc2_body(agg_ref, s_ref, b1_ref, w2_ref, t2_ref):
    a = agg_ref[0] + agg_ref[1]
    s = s_ref[...]
    h = jnp.maximum(s * a + b1_ref[...], 0.0)
    t2_ref[...] = s * jnp.dot(h, w2_ref[...],
                              preferred_element_type=jnp.float32)


def _tc3_body(agg_ref, s_ref, b2_ref, o_ref):
    a = agg_ref[0] + agg_ref[1]
    o_ref[...] = s_ref[...] * a[:, :O] + b2_ref[...]


def kernel(x, edge_index, P_vec, W1, b1, W2, b2):
    row = edge_index[0].astype(jnp.int32)
    col = edge_index[1].astype(jnp.int32)
    zpad = jnp.zeros((E_PAD - E,), jnp.int32)
    row2d = jnp.concatenate([row, zpad]).reshape(E_PAD // CHUNK, CHUNK)
    col2d = jnp.concatenate([col, zpad]).reshape(E_PAD // CHUNK, CHUNK)
    p2d = jnp.concatenate(
        [P_vec.astype(jnp.float32), jnp.zeros((E_PAD - E,), jnp.float32)]
    ).reshape(E_PAD // CHUNK, CHUNK)
    W2p = jnp.zeros((H, O_PAD), jnp.float32).at[:, :O].set(W2)

    x_pad = jnp.zeros((N_PAD, F_IN), jnp.float32).at[:N].set(x)

    w2d, degp = _deg_kernel(p2d, col2d)
    deg_t = degp.T

    t1, s = pl.pallas_call(
        _tc1_body,
        grid=(N_PAD // _BN,),
        in_specs=[
            pl.BlockSpec((_BN, F_IN), lambda i: (i, 0)),
            pl.BlockSpec((F_IN, H), lambda i: (0, 0)),
            pl.BlockSpec((_BN, 2), lambda i: (i, 0)),
        ],
        out_specs=[
            pl.BlockSpec((_BN, H), lambda i: (i, 0)),
            pl.BlockSpec((_BN, 1), lambda i: (i, 0)),
        ],
        out_shape=[
            jax.ShapeDtypeStruct((N_PAD, H), jnp.float32),
            jax.ShapeDtypeStruct((N_PAD, 1), jnp.float32),
        ],
    )(x_pad, W1, deg_t)

    agg1 = _mp1(t1, row2d, col2d, w2d)

    t2 = pl.pallas_call(
        _tc2_body,
        grid=(N_PAD // _BN,),
        in_specs=[
            pl.BlockSpec((NC, _BN, H), lambda i: (0, i, 0)),
            pl.BlockSpec((_BN, 1), lambda i: (i, 0)),
            pl.BlockSpec((1, H), lambda i: (0, 0)),
            pl.BlockSpec((H, O_PAD), lambda i: (0, 0)),
        ],
        out_specs=pl.BlockSpec((_BN, O_PAD), lambda i: (i, 0)),
        out_shape=jax.ShapeDtypeStruct((N_PAD, O_PAD), jnp.float32),
    )(agg1, s, b1.reshape(1, H), W2p)

    agg2 = _mp2(t2, row2d, col2d, w2d)

    out = pl.pallas_call(
        _tc3_body,
        grid=(N_PAD // _BN,),
        in_specs=[
            pl.BlockSpec((NC, _BN, O_PAD), lambda i: (0, i, 0)),
            pl.BlockSpec((_BN, 1), lambda i: (i, 0)),
            pl.BlockSpec((1, O), lambda i: (0, 0)),
        ],
        out_specs=pl.BlockSpec((_BN, O), lambda i: (i, 0)),
        out_shape=jax.ShapeDtypeStruct((N_PAD, O), jnp.float32),
    )(agg2, s, b2.reshape(1, O))
    return out[:N]


# speedup vs baseline: 29.0929x; 1.1876x over previous
import functools

import jax
import jax.numpy as jnp
from jax import lax
from jax.experimental import pallas as pl
from jax.experimental.pallas import tpu as pltpu
from jax.experimental.pallas import tpu_sc as plsc

N = 10000
E = 320000
F_IN = 128
H = 32
O = 8
O_PAD = 16

NC = 2
NS = 16
NW = NC * NS
CHUNK = 128
CPT = 80
EPT = CPT * CHUNK
E_PAD = NW * EPT
REAL_CHUNKS = E // CHUNK
N_PAD = 10240
NPT = N_PAD // NS
DPAD = N_PAD
DPT = DPAD // NS

_mesh = plsc.VectorSubcoreMesh(
    core_axis_name="c", subcore_axis_name="s", num_cores=NC, num_subcores=NS)


def _wid():
    return lax.axis_index("c") * NS + lax.axis_index("s")


@functools.partial(
    pl.kernel,
    out_type=(
        jax.ShapeDtypeStruct((E_PAD // CHUNK, CHUNK), jnp.float32),
        jax.ShapeDtypeStruct((NC, DPAD), jnp.float32),
    ),
    mesh=_mesh,
    compiler_params=pltpu.CompilerParams(use_tc_tiling_on_sc=False, needs_layout_passes=False),
    scratch_types=[
        pltpu.VMEM((CPT, CHUNK), jnp.float32),
        pltpu.VMEM((CPT, CHUNK), jnp.int32),
        pltpu.VMEM((CPT, CHUNK), jnp.float32),
        pltpu.VMEM((DPT,), jnp.float32),
        pltpu.VMEM_SHARED((DPAD,), jnp.float32),
    ],
)
def _deg_kernel(p2d, col2d, w2d, degp, p_v, col_v, w_v, zbuf, deg_sp):
    cid = lax.axis_index("c")
    sid = lax.axis_index("s")
    wid = cid * NS + sid
    pltpu.sync_copy(p2d.at[pl.ds(wid * CPT, CPT)], p_v)
    pltpu.sync_copy(col2d.at[pl.ds(wid * CPT, CPT)], col_v)

    def compute(i, _):
        scale = jnp.where(wid * CPT + i < REAL_CHUNKS, 1.0, 0.0)
        for c in range(CHUNK // 16):
            sl = pl.ds(c * 16, 16)
            p = p_v[i, sl]
            w_v[i, sl] = scale / (1.0 + jnp.exp(-p))
        return _
    lax.fori_loop(0, CPT, compute, None)
    pltpu.sync_copy(w_v, w2d.at[pl.ds(wid * CPT, CPT)])

    def zero(j, _):
        zbuf[pl.ds(j * 16, 16)] = jnp.zeros((16,), jnp.float32)
        return _
    lax.fori_loop(0, DPT // 16, zero, None)
    pltpu.sync_copy(zbuf, deg_sp.at[pl.ds(sid * DPT, DPT)])
    plsc.subcore_barrier()

    def scatter(i, _):
        pltpu.sync_copy(w_v.at[i], deg_sp.at[col_v.at[i]], add=True)
        return _
    lax.fori_loop(0, CPT, scatter, None)
    plsc.subcore_barrier()
    pltpu.sync_copy(deg_sp.at[pl.ds(sid * DPT, DPT)],
                    degp.at[cid, pl.ds(sid * DPT, DPT)])


NB = 4


def _make_mp_kernel(F):
    nz = 5

    @functools.partial(
        pl.kernel,
        out_type=jax.ShapeDtypeStruct((NC, N_PAD, F), jnp.float32),
        mesh=_mesh,
        compiler_params=pltpu.CompilerParams(use_tc_tiling_on_sc=False, needs_layout_passes=False),
        scratch_types=[
            pltpu.VMEM((CPT, CHUNK), jnp.int32),
            pltpu.VMEM((CPT, CHUNK), jnp.int32),
            pltpu.VMEM((CPT, CHUNK), jnp.float32),
            [pltpu.VMEM((CHUNK, F), jnp.float32) for _ in range(NB)],
            [pltpu.VMEM((CHUNK, F), jnp.float32) for _ in range(NB)],
            [pltpu.SemaphoreType.DMA for _ in range(NB)],
            [pltpu.SemaphoreType.DMA for _ in range(NB)],
            pltpu.VMEM((NPT // nz, F), jnp.float32),
            pltpu.VMEM_SHARED((N_PAD, F), jnp.float32),
        ],
    )
    def _mp(t_hbm, row2d, col2d, w2d, agg_out, row_v, col_v, w_v, gbufs,
            mbufs, gsems, ssems, zbuf, agg_sp):
        cid = lax.axis_index("c")
        sid = lax.axis_index("s")
        wid = cid * NS + sid
        pltpu.sync_copy(row2d.at[pl.ds(wid * CPT, CPT)], row_v)
        pltpu.sync_copy(col2d.at[pl.ds(wid * CPT, CPT)], col_v)
        pltpu.sync_copy(w2d.at[pl.ds(wid * CPT, CPT)], w_v)

        @pl.when(cid == 0)
        def _():
            pltpu.sync_copy(t_hbm.at[pl.ds(sid * NPT, NPT)],
                            agg_sp.at[pl.ds(sid * NPT, NPT)])

        @pl.when(cid != 0)
        def _():
            def zero(j, _):
                for c in range(F // 16):
                    zbuf[j, pl.ds(c * 16, 16)] = jnp.zeros((16,), jnp.float32)
                return _
            lax.fori_loop(0, NPT // nz, zero, None)
            for k in range(nz):
                pltpu.sync_copy(
                    zbuf, agg_sp.at[pl.ds(sid * NPT + k * (NPT // nz),
                                          NPT // nz)])
        plsc.subcore_barrier()

        for b in range(NB):
            pltpu.async_copy(t_hbm.at[row_v.at[b]], gbufs[b], gsems[b])

        def chunk_group(g, _):
            for b in range(NB):
                i = g * NB + b
                pltpu.make_async_copy(
                    t_hbm.at[row_v.at[i]], gbufs[b], gsems[b]).wait()

                @pl.when(g > 0)
                def _():
                    pltpu.make_async_copy(
                        mbufs[b], agg_sp.at[col_v.at[i]], ssems[b]).wait()
                for e in range(CHUNK):
                    bc = plsc.load_gather(
                        w_v, [jnp.full((16,), i, jnp.int32),
                              jnp.full((16,), e, jnp.int32)])
                    for c in range(F // 16):
                        sl = pl.ds(c * 16, 16)
                        mbufs[b][e, sl] = gbufs[b][e, sl] * bc

                @pl.when(i + NB < CPT)
                def _():
                    pltpu.async_copy(
                        t_hbm.at[row_v.at[i + NB]], gbufs[b], gsems[b])
                pltpu.async_copy(
                    mbufs[b], agg_sp.at[col_v.at[i]], ssems[b], add=True)
            return _
        lax.fori_loop(0, CPT // NB, chunk_group, None)
        for b in range(NB):
            pltpu.make_async_copy(
                mbufs[b], agg_sp.at[col_v.at[CPT - NB + b]], ssems[b]).wait()
        plsc.subcore_barrier()
        pltpu.sync_copy(agg_sp.at[pl.ds(sid * NPT, NPT)],
                        agg_out.at[cid, pl.ds(sid * NPT, NPT)])

    return _mp


_mp1 = _make_mp_kernel(H)
_mp2 = _make_mp_kernel(O_PAD)


_BN = 2048


def _tc1_body(x_ref, w1_ref, deg_ref, t1_ref, s_ref):
    d = deg_ref[:, 0] + deg_ref[:, 1] + 1.0
    s = lax.rsqrt(d)
    sup = jnp.dot(x_ref[...], w1_ref[...], preferred_element_type=jnp.float32)
    t1_ref[...] = s[:, None] * sup
    s_ref[...] = s[:, None]


def _tc2_body(agg_ref, s_ref, b1_ref, w2_ref, t2_ref):
    a = agg_ref[0] + agg_ref[1]
    s = s_ref[...]
    h = jnp.maximum(s * a + b1_ref[...], 0.0)
    t2_ref[...] = s * jnp.dot(h, w2_ref[...],
                              preferred_element_type=jnp.float32)


def _tc3_body(agg_ref, s_ref, b2_ref, o_ref):
    a = agg_ref[0] + agg_ref[1]
    o_ref[...] = s_ref[...] * a[:, :O] + b2_ref[...]


def kernel(x, edge_index, P_vec, W1, b1, W2, b2):
    row = edge_index[0].astype(jnp.int32)
    col = edge_index[1].astype(jnp.int32)
    zpad = jnp.zeros((E_PAD - E,), jnp.int32)
    row2d = jnp.concatenate([row, zpad]).reshape(E_PAD // CHUNK, CHUNK)
    col2d = jnp.concatenate([col, zpad]).reshape(E_PAD // CHUNK, CHUNK)
    p2d = jnp.concatenate(
        [P_vec.astype(jnp.float32), jnp.zeros((E_PAD - E,), jnp.float32)]
    ).reshape(E_PAD // CHUNK, CHUNK)
    W2p = jnp.zeros((H, O_PAD), jnp.float32).at[:, :O].set(W2)

    x_pad = jnp.zeros((N_PAD, F_IN), jnp.float32).at[:N].set(x)

    w2d, degp = _deg_kernel(p2d, col2d)
    deg_t = degp.T

    t1, s = pl.pallas_call(
        _tc1_body,
        grid=(N_PAD // _BN,),
        in_specs=[
            pl.BlockSpec((_BN, F_IN), lambda i: (i, 0)),
            pl.BlockSpec((F_IN, H), lambda i: (0, 0)),
            pl.BlockSpec((_BN, 2), lambda i: (i, 0)),
        ],
        out_specs=[
            pl.BlockSpec((_BN, H), lambda i: (i, 0)),
            pl.BlockSpec((_BN, 1), lambda i: (i, 0)),
        ],
        out_shape=[
            jax.ShapeDtypeStruct((N_PAD, H), jnp.float32),
            jax.ShapeDtypeStruct((N_PAD, 1), jnp.float32),
        ],
    )(x_pad, W1, deg_t)

    agg1 = _mp1(t1, row2d, col2d, w2d)

    t2 = pl.pallas_call(
        _tc2_body,
        grid=(N_PAD // _BN,),
        in_specs=[
            pl.BlockSpec((NC, _BN, H), lambda i: (0, i, 0)),
            pl.BlockSpec((_BN, 1), lambda i: (i, 0)),
            pl.BlockSpec((1, H), lambda i: (0, 0)),
            pl.BlockSpec((H, O_PAD), lambda i: (0, 0)),
        ],
        out_specs=pl.BlockSpec((_BN, O_PAD), lambda i: (i, 0)),
        out_shape=jax.ShapeDtypeStruct((N_PAD, O_PAD), jnp.float32),
    )(agg1, s, b1.reshape(1, H), W2p)

    agg2 = _mp2(t2, row2d, col2d, w2d)

    out = pl.pallas_call(
        _tc3_body,
        grid=(N_PAD // _BN,),
        in_specs=[
            pl.BlockSpec((NC, _BN, O_PAD), lambda i: (0, i, 0)),
            pl.BlockSpec((_BN, 1), lambda i: (i, 0)),
            pl.BlockSpec((1, O), lambda i: (0, 0)),
        ],
        out_specs=pl.BlockSpec((_BN, O), lambda i: (i, 0)),
        out_shape=jax.ShapeDtypeStruct((N_PAD, O), jnp.float32),
    )(agg2, s, b2.reshape(1, O))
    return out[:N]


# speedup vs baseline: 31.0940x; 1.0688x over previous
import functools

import jax
import jax.numpy as jnp
from jax import lax
from jax.experimental import pallas as pl
from jax.experimental.pallas import tpu as pltpu
from jax.experimental.pallas import tpu_sc as plsc

N = 10000
E = 320000
F_IN = 128
H = 32
O = 8
O_PAD = 16

NC = 2
NS = 16
NW = NC * NS
CHUNK = 128
CPT = 80
EPT = CPT * CHUNK
E_PAD = NW * EPT
REAL_CHUNKS = E // CHUNK
N_PAD = 10240
NPT = N_PAD // NS
DPAD = N_PAD
DPT = DPAD // NS

_mesh = plsc.VectorSubcoreMesh(
    core_axis_name="c", subcore_axis_name="s", num_cores=NC, num_subcores=NS)


def _wid():
    return lax.axis_index("c") * NS + lax.axis_index("s")


@functools.partial(
    pl.kernel,
    out_type=(
        jax.ShapeDtypeStruct((E_PAD // CHUNK, CHUNK), jnp.float32),
        jax.ShapeDtypeStruct((NC, DPAD), jnp.float32),
    ),
    mesh=_mesh,
    compiler_params=pltpu.CompilerParams(use_tc_tiling_on_sc=False, needs_layout_passes=False),
    scratch_types=[
        pltpu.VMEM((CPT, CHUNK), jnp.float32),
        pltpu.VMEM((CPT, CHUNK), jnp.int32),
        pltpu.VMEM((CPT, CHUNK), jnp.float32),
        pltpu.VMEM((DPT,), jnp.float32),
        pltpu.VMEM_SHARED((DPAD,), jnp.float32),
    ],
)
def _deg_kernel(p2d, col2d, w2d, degp, p_v, col_v, w_v, zbuf, deg_sp):
    cid = lax.axis_index("c")
    sid = lax.axis_index("s")
    wid = cid * NS + sid
    pltpu.sync_copy(p2d.at[pl.ds(wid * CPT, CPT)], p_v)
    pltpu.sync_copy(col2d.at[pl.ds(wid * CPT, CPT)], col_v)

    def compute(i, _):
        scale = jnp.where(wid * CPT + i < REAL_CHUNKS, 1.0, 0.0)
        for c in range(CHUNK // 16):
            sl = pl.ds(c * 16, 16)
            p = p_v[i, sl]
            w_v[i, sl] = scale / (1.0 + jnp.exp(-p))
        return _
    lax.fori_loop(0, CPT, compute, None)
    pltpu.sync_copy(w_v, w2d.at[pl.ds(wid * CPT, CPT)])

    def zero(j, _):
        zbuf[pl.ds(j * 16, 16)] = jnp.zeros((16,), jnp.float32)
        return _
    lax.fori_loop(0, DPT // 16, zero, None)
    pltpu.sync_copy(zbuf, deg_sp.at[pl.ds(sid * DPT, DPT)])
    plsc.subcore_barrier()

    def scatter(i, _):
        pltpu.sync_copy(w_v.at[i], deg_sp.at[col_v.at[i]], add=True)
        return _
    lax.fori_loop(0, CPT, scatter, None)
    plsc.subcore_barrier()
    pltpu.sync_copy(deg_sp.at[pl.ds(sid * DPT, DPT)],
                    degp.at[cid, pl.ds(sid * DPT, DPT)])


NB = 4


def _make_mp_kernel(F):
    @functools.partial(
        pl.kernel,
        out_type=jax.ShapeDtypeStruct((NC, N_PAD, F), jnp.float32),
        mesh=_mesh,
        compiler_params=pltpu.CompilerParams(use_tc_tiling_on_sc=False, needs_layout_passes=False),
        scratch_types=[
            pltpu.VMEM((CPT, CHUNK), jnp.int32),
            pltpu.VMEM((CPT, CHUNK), jnp.int32),
            pltpu.VMEM((CPT, CHUNK), jnp.float32),
            [pltpu.VMEM((CHUNK, F), jnp.float32) for _ in range(NB)],
            [pltpu.VMEM((CHUNK, F), jnp.float32) for _ in range(NB)],
            [pltpu.SemaphoreType.DMA for _ in range(NB)],
            [pltpu.SemaphoreType.DMA for _ in range(NB)],
            pltpu.VMEM_SHARED((N_PAD, F), jnp.float32),
        ],
    )
    def _mp(t_hbm, row2d, col2d, w2d, agg_out, row_v, col_v, w_v, gbufs,
            mbufs, gsems, ssems, agg_sp):
        cid = lax.axis_index("c")
        sid = lax.axis_index("s")
        wid = cid * NS + sid
        pltpu.sync_copy(row2d.at[pl.ds(wid * CPT, CPT)], row_v)
        pltpu.sync_copy(col2d.at[pl.ds(wid * CPT, CPT)], col_v)
        pltpu.sync_copy(w2d.at[pl.ds(wid * CPT, CPT)], w_v)

        pltpu.sync_copy(t_hbm.at[pl.ds(sid * NPT, NPT)],
                        agg_sp.at[pl.ds(sid * NPT, NPT)])
        plsc.subcore_barrier()

        def multiply(b, i):
            if F >= 16:
                for e in range(CHUNK):
                    bc = plsc.load_gather(
                        w_v, [jnp.full((16,), i, jnp.int32),
                              jnp.full((16,), e, jnp.int32)])
                    for c in range(F // 16):
                        sl = pl.ds(c * 16, 16)
                        mbufs[b][e, sl] = gbufs[b][e, sl] * bc
            else:
                epl = 16 // F
                erow = lax.iota(jnp.int32, 16) // F
                ecol = lax.iota(jnp.int32, 16) % F
                iv = jnp.full((16,), i, jnp.int32)
                for v in range(CHUNK // epl):
                    rows = erow + (epl * v)
                    wv = plsc.load_gather(w_v, [iv, rows])
                    g = plsc.load_gather(gbufs[b], [rows, ecol])
                    plsc.store_scatter(mbufs[b], [rows, ecol], g * wv)

        for b in range(NB):
            pltpu.async_copy(t_hbm.at[row_v.at[b]], gbufs[b], gsems[b])

        def chunk_group(g, _):
            for b in range(NB):
                i = g * NB + b
                pltpu.make_async_copy(
                    t_hbm.at[row_v.at[i]], gbufs[b], gsems[b]).wait()

                @pl.when(g > 0)
                def _():
                    pltpu.make_async_copy(
                        mbufs[b], agg_sp.at[col_v.at[i]], ssems[b]).wait()
                multiply(b, i)

                @pl.when(i + NB < CPT)
                def _():
                    pltpu.async_copy(
                        t_hbm.at[row_v.at[i + NB]], gbufs[b], gsems[b])
                pltpu.async_copy(
                    mbufs[b], agg_sp.at[col_v.at[i]], ssems[b], add=True)
            return _
        lax.fori_loop(0, CPT // NB, chunk_group, None)
        for b in range(NB):
            pltpu.make_async_copy(
                mbufs[b], agg_sp.at[col_v.at[CPT - NB + b]], ssems[b]).wait()
        plsc.subcore_barrier()
        pltpu.sync_copy(agg_sp.at[pl.ds(sid * NPT, NPT)],
                        agg_out.at[cid, pl.ds(sid * NPT, NPT)])

    return _mp


_mp1 = _make_mp_kernel(H)
_mp2 = _make_mp_kernel(O)


_BN = 2048


def _tc1_body(x_ref, w1_ref, deg_ref, t1_ref, s_ref):
    d = deg_ref[:, 0] + deg_ref[:, 1] + 1.0
    s = lax.rsqrt(d)
    sup = jnp.dot(x_ref[...], w1_ref[...], preferred_element_type=jnp.float32)
    t1_ref[...] = s[:, None] * sup
    s_ref[...] = s[:, None]


def _tc2_body(agg_ref, t1_ref, s_ref, b1_ref, w2_ref, t2_ref):
    a = agg_ref[0] + agg_ref[1] - t1_ref[...]
    s = s_ref[...]
    h = jnp.maximum(s * a + b1_ref[...], 0.0)
    t2_ref[...] = s * jnp.dot(h, w2_ref[...],
                              preferred_element_type=jnp.float32)


def _tc3_body(agg_ref, t2_ref, s_ref, b2_ref, o_ref):
    a = agg_ref[0] + agg_ref[1] - t2_ref[...]
    o_ref[...] = s_ref[...] * a + b2_ref[...]


def kernel(x, edge_index, P_vec, W1, b1, W2, b2):
    row = edge_index[0].astype(jnp.int32)
    col = edge_index[1].astype(jnp.int32)
    zpad = jnp.zeros((E_PAD - E,), jnp.int32)
    row2d = jnp.concatenate([row, zpad]).reshape(E_PAD // CHUNK, CHUNK)
    col2d = jnp.concatenate([col, zpad]).reshape(E_PAD // CHUNK, CHUNK)
    p2d = jnp.concatenate(
        [P_vec.astype(jnp.float32), jnp.zeros((E_PAD - E,), jnp.float32)]
    ).reshape(E_PAD // CHUNK, CHUNK)

    x_pad = jnp.zeros((N_PAD, F_IN), jnp.float32).at[:N].set(x)

    w2d, degp = _deg_kernel(p2d, col2d)
    deg_t = degp.T

    t1, s = pl.pallas_call(
        _tc1_body,
        grid=(N_PAD // _BN,),
        in_specs=[
            pl.BlockSpec((_BN, F_IN), lambda i: (i, 0)),
            pl.BlockSpec((F_IN, H), lambda i: (0, 0)),
            pl.BlockSpec((_BN, 2), lambda i: (i, 0)),
        ],
        out_specs=[
            pl.BlockSpec((_BN, H), lambda i: (i, 0)),
            pl.BlockSpec((_BN, 1), lambda i: (i, 0)),
        ],
        out_shape=[
            jax.ShapeDtypeStruct((N_PAD, H), jnp.float32),
            jax.ShapeDtypeStruct((N_PAD, 1), jnp.float32),
        ],
    )(x_pad, W1, deg_t)

    agg1 = _mp1(t1, row2d, col2d, w2d)

    t2 = pl.pallas_call(
        _tc2_body,
        grid=(N_PAD // _BN,),
        in_specs=[
            pl.BlockSpec((NC, _BN, H), lambda i: (0, i, 0)),
            pl.BlockSpec((_BN, H), lambda i: (i, 0)),
            pl.BlockSpec((_BN, 1), lambda i: (i, 0)),
            pl.BlockSpec((1, H), lambda i: (0, 0)),
            pl.BlockSpec((H, O), lambda i: (0, 0)),
        ],
        out_specs=pl.BlockSpec((_BN, O), lambda i: (i, 0)),
        out_shape=jax.ShapeDtypeStruct((N_PAD, O), jnp.float32),
    )(agg1, t1, s, b1.reshape(1, H), W2)

    agg2 = _mp2(t2, row2d, col2d, w2d)

    out = pl.pallas_call(
        _tc3_body,
        grid=(N_PAD // _BN,),
        in_specs=[
            pl.BlockSpec((NC, _BN, O), lambda i: (0, i, 0)),
            pl.BlockSpec((_BN, O), lambda i: (i, 0)),
            pl.BlockSpec((_BN, 1), lambda i: (i, 0)),
            pl.BlockSpec((1, O), lambda i: (0, 0)),
        ],
        out_specs=pl.BlockSpec((_BN, O), lambda i: (i, 0)),
        out_shape=jax.ShapeDtypeStruct((N_PAD, O), jnp.float32),
    )(agg2, t2, s, b2.reshape(1, O))
    return out[:N]
